# baseline (speedup 1.0000x reference)
import dataclasses
import functools

import jax
import jax.numpy as jnp
from jax import lax
from jax.experimental import pallas as pl
from jax.experimental.pallas import tpu as pltpu
from jax.experimental.pallas import tpu_sc as plsc

N = 10000
E = 320000
D = 128
EPS = 1e-5

NC = 2
NS = 16
EPT = E // (NC * NS)
BLK = 128
NBLK = 80
EPTP = NBLK * BLK
NP = 10240
RPT = NP // NS
ZROWS = 32
LANES = 16

_mesh = plsc.VectorSubcoreMesh(core_axis_name="c", subcore_axis_name="s",
                               num_cores=NC)

_sc_params = pltpu.CompilerParams()
if "needs_layout_passes" in pltpu.CompilerParams.__dataclass_fields__:
    _sc_params = dataclasses.replace(_sc_params, needs_layout_passes=False)


def _make_sc_agg():
    @functools.partial(
        pl.kernel,
        out_type=jax.ShapeDtypeStruct((NC, NP, D), jnp.float32),
        mesh=_mesh,
        compiler_params=_sc_params,
        scratch_types=[
            pltpu.VMEM((3, BLK), jnp.int32),
            pltpu.VMEM((3, BLK), jnp.int32),
            pltpu.VMEM((BLK, D), jnp.float32),
            pltpu.VMEM((BLK, D), jnp.float32),
            pltpu.VMEM((ZROWS, D), jnp.float32),
            pltpu.VMEM_SHARED((NP, D), jnp.float32),
            pltpu.SemaphoreType.DMA,
            pltpu.SemaphoreType.DMA,
        ],
    )
    def k(table_hbm, edata_hbm, out_hbm,
          ebufa, ebufb, rowsa, rowsb, zbuf, acc,
          gsema, gsemb):
        c = lax.axis_index("c")
        s = lax.axis_index("s")

        @pl.loop(0, ZROWS)
        def _(r):
            for j in range(D // LANES):
                zbuf[r, pl.ds(j * LANES, LANES)] = jnp.zeros(
                    (LANES,), jnp.float32)

        @pl.loop(0, RPT // ZROWS)
        def _(kk):
            pltpu.sync_copy(
                zbuf, acc.at[pl.ds(s * RPT + kk * ZROWS, ZROWS)])

        plsc.subcore_barrier()

        def scale(ebuf, rows):
            @pl.loop(0, BLK // LANES)
            def _(g):
                wchunk = plsc.bitcast(
                    ebuf[2, pl.ds(g * LANES, LANES)], jnp.float32)
                for k2 in range(LANES):
                    w_i = wchunk[k2]
                    i = g * LANES + k2
                    for j in range(D // LANES):
                        sl = pl.ds(j * LANES, LANES)
                        rows[i, sl] = rows[i, sl] * w_i

        pltpu.sync_copy(edata_hbm.at[c, s, 0], ebufa)
        ga = pltpu.async_copy(table_hbm.at[ebufa.at[0]], rowsa, gsema)
        pltpu.sync_copy(edata_hbm.at[c, s, 1], ebufb)
        gb = pltpu.async_copy(table_hbm.at[ebufb.at[0]], rowsb, gsemb)

        @pl.loop(0, NBLK // 2 - 1)
        def _(p):
            ga.wait()
            scale(ebufa, rowsa)
            sa = pltpu.async_copy(rowsa, acc.at[ebufa.at[1]], gsema,
                                  add=True)
            gb.wait()
            scale(ebufb, rowsb)
            sb = pltpu.async_copy(rowsb, acc.at[ebufb.at[1]], gsemb,
                                  add=True)
            sa.wait()
            pltpu.sync_copy(edata_hbm.at[c, s, 2 * p + 2], ebufa)
            pltpu.async_copy(table_hbm.at[ebufa.at[0]], rowsa, gsema)
            sb.wait()
            pltpu.sync_copy(edata_hbm.at[c, s, 2 * p + 3], ebufb)
            pltpu.async_copy(table_hbm.at[ebufb.at[0]], rowsb, gsemb)

        ga.wait()
        scale(ebufa, rowsa)
        pltpu.sync_copy(rowsa, acc.at[ebufa.at[1]], add=True)
        gb.wait()
        scale(ebufb, rowsb)
        pltpu.sync_copy(rowsb, acc.at[ebufb.at[1]], add=True)

        plsc.subcore_barrier()

        pltpu.sync_copy(acc.at[pl.ds(s * RPT, RPT)],
                        out_hbm.at[c, pl.ds(s * RPT, RPT)])

    return k


_sc_agg = _make_sc_agg()


def _matmul_t(a, w):
    def body(a_ref, w_ref, o_ref):
        o_ref[...] = lax.dot_general(
            a_ref[...], w_ref[...], (((1,), (1,)), ((), ())),
            preferred_element_type=jnp.float32,
            precision=lax.Precision.HIGHEST)

    return pl.pallas_call(
        body,
        out_shape=jax.ShapeDtypeStruct((a.shape[0], w.shape[0]), jnp.float32),
    )(a, w)


def _combine_bn_tanh(aggp, root, w_rel, b, gamma, beta):
    def body(p_ref, r_ref, w_ref, b_ref, g_ref, be_ref, o_ref):
        agg = p_ref[0, pl.ds(0, N)] + p_ref[1, pl.ds(0, N)]
        y = lax.dot_general(
            agg, w_ref[...], (((1,), (1,)), ((), ())),
            preferred_element_type=jnp.float32,
            precision=lax.Precision.HIGHEST)
        y = y + b_ref[...] + r_ref[...]
        mean = jnp.mean(y, axis=0, keepdims=True)
        var = jnp.mean((y - mean) ** 2, axis=0, keepdims=True)
        xn = (y - mean) * lax.rsqrt(var + EPS)
        o_ref[...] = jnp.tanh(xn * g_ref[...] + be_ref[...])

    return pl.pallas_call(
        body,
        out_shape=jax.ShapeDtypeStruct((N, D), jnp.float32),
    )(aggp, root, w_rel, b, gamma, beta)


def _combine(aggp, root, w_rel, b):
    def body(p_ref, r_ref, w_ref, b_ref, o_ref):
        agg = p_ref[0, pl.ds(0, N)] + p_ref[1, pl.ds(0, N)]
        y = lax.dot_general(
            agg, w_ref[...], (((1,), (1,)), ((), ())),
            preferred_element_type=jnp.float32,
            precision=lax.Precision.HIGHEST)
        o_ref[...] = y + b_ref[...] + r_ref[...]

    return pl.pallas_call(
        body,
        out_shape=jax.ShapeDtypeStruct((N, D), jnp.float32),
    )(aggp, root, w_rel, b)


def kernel(x, edge_index, edge_attr, W1_rel, b1_rel, W1_root, gamma, beta,
           W2_rel, b2_rel, W2_root):
    pad = ((0, 0), (0, 0), (0, EPTP - EPT))
    src3 = jnp.pad(edge_index[0].astype(jnp.int32).reshape(NC, NS, EPT), pad,
                   constant_values=0).reshape(NC, NS, NBLK, 1, BLK)
    dst3 = jnp.pad(edge_index[1].astype(jnp.int32).reshape(NC, NS, EPT), pad,
                   constant_values=NP - 1).reshape(NC, NS, NBLK, 1, BLK)
    wbits = jnp.pad(jax.lax.bitcast_convert_type(
        edge_attr.astype(jnp.float32), jnp.int32).reshape(NC, NS, EPT), pad,
        constant_values=0).reshape(NC, NS, NBLK, 1, BLK)
    edata = jnp.concatenate([src3, dst3, wbits], axis=3)

    b1 = b1_rel.reshape(1, D)
    b2 = b2_rel.reshape(1, D)
    g = gamma.reshape(1, D)
    be = beta.reshape(1, D)

    p1 = _sc_agg(x, edata)
    xr = _matmul_t(x, W1_root)
    h = _combine_bn_tanh(p1, xr, W1_rel, b1, g, be)

    p2 = _sc_agg(h, edata)
    hr = _matmul_t(h, W2_root)
    out = _combine(p2, hr, W2_rel, b2)
    return out


# speedup vs baseline: 1.1274x; 1.1274x over previous
import dataclasses
import functools

import jax
import jax.numpy as jnp
from jax import lax
from jax.experimental import pallas as pl
from jax.experimental.pallas import tpu as pltpu
from jax.experimental.pallas import tpu_sc as plsc

N = 10000
E = 320000
D = 128
EPS = 1e-5

NC = 2
NS = 16
EPT = E // (NC * NS)
BLK = 32
NBLK = 314
EPTP = NBLK * BLK
NP = 10240
RPT = NP // NS
ZROWS = 32
LANES = 16

_mesh = plsc.VectorSubcoreMesh(core_axis_name="c", subcore_axis_name="s",
                               num_cores=NC)

_sc_params = pltpu.CompilerParams()
if "needs_layout_passes" in pltpu.CompilerParams.__dataclass_fields__:
    _sc_params = dataclasses.replace(_sc_params, needs_layout_passes=False)


def _make_sc_agg():
    @functools.partial(
        pl.kernel,
        out_type=jax.ShapeDtypeStruct((NC, NP, D), jnp.float32),
        mesh=_mesh,
        compiler_params=_sc_params,
        scratch_types=[
            pltpu.VMEM((3, BLK), jnp.int32),
            pltpu.VMEM((3, BLK), jnp.int32),
            pltpu.VMEM((BLK, D), jnp.float32),
            pltpu.VMEM((BLK, D), jnp.float32),
            pltpu.VMEM((ZROWS, D), jnp.float32),
            pltpu.VMEM_SHARED((NP, D), jnp.float32),
            pltpu.SemaphoreType.DMA,
            pltpu.SemaphoreType.DMA,
        ],
    )
    def k(table_hbm, edata_hbm, out_hbm,
          ebufa, ebufb, rowsa, rowsb, zbuf, acc,
          gsema, gsemb):
        c = lax.axis_index("c")
        s = lax.axis_index("s")

        @pl.loop(0, ZROWS)
        def _(r):
            for j in range(D // LANES):
                zbuf[r, pl.ds(j * LANES, LANES)] = jnp.zeros(
                    (LANES,), jnp.float32)

        @pl.loop(0, RPT // ZROWS)
        def _(kk):
            pltpu.sync_copy(
                zbuf, acc.at[pl.ds(s * RPT + kk * ZROWS, ZROWS)])

        plsc.subcore_barrier()

        def scale(ebuf, rows):
            @pl.loop(0, BLK // LANES)
            def _(g):
                wchunk = plsc.bitcast(
                    ebuf[2, pl.ds(g * LANES, LANES)], jnp.float32)
                for k2 in range(LANES):
                    w_i = wchunk[k2]
                    i = g * LANES + k2
                    for j in range(D // LANES):
                        sl = pl.ds(j * LANES, LANES)
                        rows[i, sl] = rows[i, sl] * w_i

        pltpu.sync_copy(edata_hbm.at[c, s, 0], ebufa)
        ga = pltpu.async_copy(table_hbm.at[ebufa.at[0]], rowsa, gsema)
        pltpu.sync_copy(edata_hbm.at[c, s, 1], ebufb)
        gb = pltpu.async_copy(table_hbm.at[ebufb.at[0]], rowsb, gsemb)

        @pl.loop(0, NBLK // 2 - 1)
        def _(p):
            ga.wait()
            scale(ebufa, rowsa)
            sa = pltpu.async_copy(rowsa, acc.at[ebufa.at[1]], gsema,
                                  add=True)
            gb.wait()
            scale(ebufb, rowsb)
            sb = pltpu.async_copy(rowsb, acc.at[ebufb.at[1]], gsemb,
                                  add=True)
            sa.wait()
            pltpu.sync_copy(edata_hbm.at[c, s, 2 * p + 2], ebufa)
            pltpu.async_copy(table_hbm.at[ebufa.at[0]], rowsa, gsema)
            sb.wait()
            pltpu.sync_copy(edata_hbm.at[c, s, 2 * p + 3], ebufb)
            pltpu.async_copy(table_hbm.at[ebufb.at[0]], rowsb, gsemb)

        ga.wait()
        scale(ebufa, rowsa)
        pltpu.sync_copy(rowsa, acc.at[ebufa.at[1]], add=True)
        gb.wait()
        scale(ebufb, rowsb)
        pltpu.sync_copy(rowsb, acc.at[ebufb.at[1]], add=True)

        plsc.subcore_barrier()

        pltpu.sync_copy(acc.at[pl.ds(s * RPT, RPT)],
                        out_hbm.at[c, pl.ds(s * RPT, RPT)])

    return k


_sc_agg = _make_sc_agg()


def _matmul_t(a, w):
    def body(a_ref, w_ref, o_ref):
        o_ref[...] = lax.dot_general(
            a_ref[...], w_ref[...], (((1,), (1,)), ((), ())),
            preferred_element_type=jnp.float32,
            precision=lax.Precision.HIGHEST)

    return pl.pallas_call(
        body,
        out_shape=jax.ShapeDtypeStruct((a.shape[0], w.shape[0]), jnp.float32),
    )(a, w)


def _combine_bn_tanh(aggp, root, w_rel, b, gamma, beta):
    def body(p_ref, r_ref, w_ref, b_ref, g_ref, be_ref, o_ref):
        agg = p_ref[0, pl.ds(0, N)] + p_ref[1, pl.ds(0, N)]
        y = lax.dot_general(
            agg, w_ref[...], (((1,), (1,)), ((), ())),
            preferred_element_type=jnp.float32,
            precision=lax.Precision.HIGHEST)
        y = y + b_ref[...] + r_ref[...]
        mean = jnp.mean(y, axis=0, keepdims=True)
        var = jnp.mean((y - mean) ** 2, axis=0, keepdims=True)
        xn = (y - mean) * lax.rsqrt(var + EPS)
        o_ref[...] = jnp.tanh(xn * g_ref[...] + be_ref[...])

    return pl.pallas_call(
        body,
        out_shape=jax.ShapeDtypeStruct((N, D), jnp.float32),
    )(aggp, root, w_rel, b, gamma, beta)


def _combine(aggp, root, w_rel, b):
    def body(p_ref, r_ref, w_ref, b_ref, o_ref):
        agg = p_ref[0, pl.ds(0, N)] + p_ref[1, pl.ds(0, N)]
        y = lax.dot_general(
            agg, w_ref[...], (((1,), (1,)), ((), ())),
            preferred_element_type=jnp.float32,
            precision=lax.Precision.HIGHEST)
        o_ref[...] = y + b_ref[...] + r_ref[...]

    return pl.pallas_call(
        body,
        out_shape=jax.ShapeDtypeStruct((N, D), jnp.float32),
    )(aggp, root, w_rel, b)


def kernel(x, edge_index, edge_attr, W1_rel, b1_rel, W1_root, gamma, beta,
           W2_rel, b2_rel, W2_root):
    pad = ((0, 0), (0, 0), (0, EPTP - EPT))
    src3 = jnp.pad(edge_index[0].astype(jnp.int32).reshape(NC, NS, EPT), pad,
                   constant_values=0).reshape(NC, NS, NBLK, 1, BLK)
    dst3 = jnp.pad(edge_index[1].astype(jnp.int32).reshape(NC, NS, EPT), pad,
                   constant_values=NP - 1).reshape(NC, NS, NBLK, 1, BLK)
    wbits = jnp.pad(jax.lax.bitcast_convert_type(
        edge_attr.astype(jnp.float32), jnp.int32).reshape(NC, NS, EPT), pad,
        constant_values=0).reshape(NC, NS, NBLK, 1, BLK)
    edata = jnp.concatenate([src3, dst3, wbits], axis=3)

    b1 = b1_rel.reshape(1, D)
    b2 = b2_rel.reshape(1, D)
    g = gamma.reshape(1, D)
    be = beta.reshape(1, D)

    p1 = _sc_agg(x, edata)
    xr = _matmul_t(x, W1_root)
    h = _combine_bn_tanh(p1, xr, W1_rel, b1, g, be)

    p2 = _sc_agg(h, edata)
    hr = _matmul_t(h, W2_root)
    out = _combine(p2, hr, W2_rel, b2)
    return out


# speedup vs baseline: 1.2652x; 1.1223x over previous
import dataclasses
import functools

import jax
import jax.numpy as jnp
from jax import lax
from jax.experimental import pallas as pl
from jax.experimental.pallas import tpu as pltpu
from jax.experimental.pallas import tpu_sc as plsc

N = 10000
E = 320000
D = 128
EPS = 1e-5

NC = 2
NS = 16
EPT = E // (NC * NS)
BLK = 64
NBLK = 158
EPTP = NBLK * BLK
NP = 10240
RPT = NP // NS
ZROWS = 32
LANES = 16

_mesh = plsc.VectorSubcoreMesh(core_axis_name="c", subcore_axis_name="s",
                               num_cores=NC)

_sc_params = pltpu.CompilerParams()
if "needs_layout_passes" in pltpu.CompilerParams.__dataclass_fields__:
    _sc_params = dataclasses.replace(_sc_params, needs_layout_passes=False)


def _make_sc_agg():
    @functools.partial(
        pl.kernel,
        out_type=jax.ShapeDtypeStruct((NC, NP, D), jnp.float32),
        mesh=_mesh,
        compiler_params=_sc_params,
        scratch_types=[
            pltpu.VMEM((3, BLK), jnp.int32),
            pltpu.VMEM((3, BLK), jnp.int32),
            pltpu.VMEM((BLK, D), jnp.float32),
            pltpu.VMEM((BLK, D), jnp.float32),
            pltpu.VMEM((ZROWS, D), jnp.float32),
            pltpu.VMEM_SHARED((NP, D), jnp.float32),
            pltpu.SemaphoreType.DMA,
            pltpu.SemaphoreType.DMA,
        ],
    )
    def k(table_hbm, edata_hbm, out_hbm,
          ebufa, ebufb, rowsa, rowsb, zbuf, acc,
          gsema, gsemb):
        c = lax.axis_index("c")
        s = lax.axis_index("s")

        @pl.loop(0, ZROWS)
        def _(r):
            for j in range(D // LANES):
                zbuf[r, pl.ds(j * LANES, LANES)] = jnp.zeros(
                    (LANES,), jnp.float32)

        @pl.loop(0, RPT // ZROWS)
        def _(kk):
            pltpu.sync_copy(
                zbuf, acc.at[pl.ds(s * RPT + kk * ZROWS, ZROWS)])

        plsc.subcore_barrier()

        def scale(ebuf, rows):
            @pl.loop(0, BLK // LANES)
            def _(g):
                wchunk = plsc.bitcast(
                    ebuf[2, pl.ds(g * LANES, LANES)], jnp.float32)
                for k2 in range(LANES):
                    w_i = wchunk[k2]
                    i = g * LANES + k2
                    for j in range(D // LANES):
                        sl = pl.ds(j * LANES, LANES)
                        rows[i, sl] = rows[i, sl] * w_i

        pltpu.sync_copy(edata_hbm.at[c, s, 0], ebufa)
        ga = pltpu.async_copy(table_hbm.at[ebufa.at[0]], rowsa, gsema)
        pltpu.sync_copy(edata_hbm.at[c, s, 1], ebufb)
        gb = pltpu.async_copy(table_hbm.at[ebufb.at[0]], rowsb, gsemb)

        @pl.loop(0, NBLK // 2 - 1)
        def _(p):
            ga.wait()
            scale(ebufa, rowsa)
            sa = pltpu.async_copy(rowsa, acc.at[ebufa.at[1]], gsema,
                                  add=True)
            gb.wait()
            scale(ebufb, rowsb)
            sb = pltpu.async_copy(rowsb, acc.at[ebufb.at[1]], gsemb,
                                  add=True)
            sa.wait()
            pltpu.sync_copy(edata_hbm.at[c, s, 2 * p + 2], ebufa)
            pltpu.async_copy(table_hbm.at[ebufa.at[0]], rowsa, gsema)
            sb.wait()
            pltpu.sync_copy(edata_hbm.at[c, s, 2 * p + 3], ebufb)
            pltpu.async_copy(table_hbm.at[ebufb.at[0]], rowsb, gsemb)

        ga.wait()
        scale(ebufa, rowsa)
        pltpu.sync_copy(rowsa, acc.at[ebufa.at[1]], add=True)
        gb.wait()
        scale(ebufb, rowsb)
        pltpu.sync_copy(rowsb, acc.at[ebufb.at[1]], add=True)

        plsc.subcore_barrier()

        pltpu.sync_copy(acc.at[pl.ds(s * RPT, RPT)],
                        out_hbm.at[c, pl.ds(s * RPT, RPT)])

    return k


_sc_agg = _make_sc_agg()


def _matmul_t(a, w):
    def body(a_ref, w_ref, o_ref):
        o_ref[...] = lax.dot_general(
            a_ref[...], w_ref[...], (((1,), (1,)), ((), ())),
            preferred_element_type=jnp.float32,
            precision=lax.Precision.HIGHEST)

    return pl.pallas_call(
        body,
        out_shape=jax.ShapeDtypeStruct((a.shape[0], w.shape[0]), jnp.float32),
    )(a, w)


def _combine_bn_tanh(aggp, root, w_rel, b, gamma, beta):
    def body(p_ref, r_ref, w_ref, b_ref, g_ref, be_ref, o_ref):
        agg = p_ref[0, pl.ds(0, N)] + p_ref[1, pl.ds(0, N)]
        y = lax.dot_general(
            agg, w_ref[...], (((1,), (1,)), ((), ())),
            preferred_element_type=jnp.float32,
            precision=lax.Precision.HIGHEST)
        y = y + b_ref[...] + r_ref[...]
        mean = jnp.mean(y, axis=0, keepdims=True)
        var = jnp.mean((y - mean) ** 2, axis=0, keepdims=True)
        xn = (y - mean) * lax.rsqrt(var + EPS)
        o_ref[...] = jnp.tanh(xn * g_ref[...] + be_ref[...])

    return pl.pallas_call(
        body,
        out_shape=jax.ShapeDtypeStruct((N, D), jnp.float32),
    )(aggp, root, w_rel, b, gamma, beta)


def _combine(aggp, root, w_rel, b):
    def body(p_ref, r_ref, w_ref, b_ref, o_ref):
        agg = p_ref[0, pl.ds(0, N)] + p_ref[1, pl.ds(0, N)]
        y = lax.dot_general(
            agg, w_ref[...], (((1,), (1,)), ((), ())),
            preferred_element_type=jnp.float32,
            precision=lax.Precision.HIGHEST)
        o_ref[...] = y + b_ref[...] + r_ref[...]

    return pl.pallas_call(
        body,
        out_shape=jax.ShapeDtypeStruct((N, D), jnp.float32),
    )(aggp, root, w_rel, b)


def kernel(x, edge_index, edge_attr, W1_rel, b1_rel, W1_root, gamma, beta,
           W2_rel, b2_rel, W2_root):
    pad = ((0, 0), (0, 0), (0, EPTP - EPT))
    src3 = jnp.pad(edge_index[0].astype(jnp.int32).reshape(NC, NS, EPT), pad,
                   constant_values=0).reshape(NC, NS, NBLK, 1, BLK)
    dst3 = jnp.pad(edge_index[1].astype(jnp.int32).reshape(NC, NS, EPT), pad,
                   constant_values=NP - 1).reshape(NC, NS, NBLK, 1, BLK)
    wbits = jnp.pad(jax.lax.bitcast_convert_type(
        edge_attr.astype(jnp.float32), jnp.int32).reshape(NC, NS, EPT), pad,
        constant_values=0).reshape(NC, NS, NBLK, 1, BLK)
    edata = jnp.concatenate([src3, dst3, wbits], axis=3)

    b1 = b1_rel.reshape(1, D)
    b2 = b2_rel.reshape(1, D)
    g = gamma.reshape(1, D)
    be = beta.reshape(1, D)

    p1 = _sc_agg(x, edata)
    xr = _matmul_t(x, W1_root)
    h = _combine_bn_tanh(p1, xr, W1_rel, b1, g, be)

    p2 = _sc_agg(h, edata)
    hr = _matmul_t(h, W2_root)
    out = _combine(p2, hr, W2_rel, b2)
    return out
